# baseline (speedup 1.0000x reference)
import functools

import jax
import jax.numpy as jnp
from jax import lax
from jax.experimental import pallas as pl
from jax.experimental.pallas import tpu as pltpu
from jax.experimental.pallas import tpu_sc as plsc

T, D, E, K = 2048, 768, 64, 2
B = 64
NB = T * K // B + E
P = NB * B

NC, NS = 2, 16
NW = NC * NS

_SC_MESH = dict(core_axis_name="c", subcore_axis_name="s",
                num_cores=NC, num_subcores=NS)


def _gating_body(x_ref, wg_ref, bg_ref, pslot_ref, w_ref, blk_ref, aux_ref):
    x = x_ref[...]
    logits = jnp.dot(x, wg_ref[...], preferred_element_type=jnp.float32)
    logits = logits + bg_ref[...]

    m0 = jnp.max(logits, axis=1, keepdims=True)
    ex = jnp.exp(logits - m0)
    gates = ex / jnp.sum(ex, axis=1, keepdims=True)
    imp = jnp.mean(gates, axis=0, keepdims=True)
    aux_ref[...] = jnp.sum((1.0 / E) * (jnp.log(1.0 / E) - jnp.log(imp)),
                           keepdims=True)

    lanes = lax.broadcasted_iota(jnp.int32, (T, E), 1)
    a0 = jnp.min(jnp.where(logits == m0, lanes, E), axis=1)
    masked = jnp.where(lanes == a0[:, None], -jnp.inf, logits)
    m1 = jnp.max(masked, axis=1, keepdims=True)
    a1 = jnp.min(jnp.where(masked == m1, lanes, E), axis=1)
    t = jnp.exp(m1 - m0)
    w0 = 1.0 / (1.0 + t)
    w_ref[0, :] = w0[:, 0]
    w_ref[1, :] = (t * w0)[:, 0]

    oh0 = (lanes == a0[:, None]).astype(jnp.float32)
    oh1 = (lanes == a1[:, None]).astype(jnp.float32)
    tri = (lax.broadcasted_iota(jnp.int32, (T, T), 0)
           > lax.broadcasted_iota(jnp.int32, (T, T), 1)).astype(jnp.float32)
    c0 = jnp.dot(tri, oh0, preferred_element_type=jnp.float32)
    c1 = jnp.dot(tri, oh1, preferred_element_type=jnp.float32)
    r0 = jnp.sum(c0 * oh0, axis=1)
    r1 = jnp.sum(c1 * oh1, axis=1)
    n0 = jnp.sum(oh0, axis=0, keepdims=True)
    n1 = jnp.sum(oh1, axis=0, keepdims=True)
    counts = (n0 + n1).astype(jnp.int32)
    pcounts = ((counts + (B - 1)) // B) * B
    etri = (lax.broadcasted_iota(jnp.int32, (E, E), 0)
            < lax.broadcasted_iota(jnp.int32, (E, E), 1)).astype(jnp.float32)
    poff = jnp.dot(pcounts.astype(jnp.float32), etri,
                   preferred_element_type=jnp.float32)
    poff_a0 = jnp.sum(poff * oh0, axis=1)
    poff_a1 = jnp.sum(poff * oh1, axis=1)
    n0_a1 = jnp.sum(n0 * oh1, axis=1)
    pslot_ref[0, :] = (poff_a0 + r0).astype(jnp.int32)
    pslot_ref[1, :] = (poff_a1 + n0_a1 + r1).astype(jnp.int32)

    bidx = (lax.broadcasted_iota(jnp.int32, (NB, E), 0) * B).astype(
        jnp.float32)
    blk_ref[0, :] = jnp.sum((poff <= bidx).astype(jnp.int32), axis=1) - 1


def _gating(x, Wg, bg):
    return pl.pallas_call(
        _gating_body,
        out_shape=[
            jax.ShapeDtypeStruct((2, T), jnp.int32),
            jax.ShapeDtypeStruct((2, T), jnp.float32),
            jax.ShapeDtypeStruct((1, NB), jnp.int32),
            jax.ShapeDtypeStruct((1, 1), jnp.float32),
        ],
    )(x, Wg, bg.reshape(1, E))


_APW = T * K // NW


def _dispatch_body(x_hbm, pslot_hbm, xs_hbm, idx_v, rows_v, sem):
    wid = lax.axis_index("s") * NC + lax.axis_index("c")
    base = wid * _APW
    tok0 = pl.multiple_of(base & (T - 1), 8)
    pltpu.sync_copy(pslot_hbm.at[pl.ds(base, _APW)], idx_v.at[0])
    pltpu.sync_copy(x_hbm.at[pl.ds(tok0, _APW)], rows_v)
    pltpu.async_copy(rows_v, xs_hbm.at[idx_v.at[0]], sem).wait()


def _dispatch(x, pslot_flat):
    return pl.kernel(
        _dispatch_body,
        out_type=jax.ShapeDtypeStruct((P, D), jnp.float32),
        mesh=plsc.VectorSubcoreMesh(**_SC_MESH),
        scratch_types=[
            pltpu.VMEM((1, _APW), jnp.int32),
            pltpu.VMEM((_APW, D), jnp.float32),
            pltpu.SemaphoreType.DMA,
        ],
    )(x, pslot_flat)


def _expert_body(blk_e_ref, xs_ref, we_ref, be_ref, ys_ref):
    del blk_e_ref
    acc = jnp.dot(xs_ref[...].astype(jnp.bfloat16), we_ref[0],
                  preferred_element_type=jnp.float32)
    ys_ref[...] = acc + be_ref[0]


def _expert_matmul(blk_e, xs, We, be):
    grid_spec = pltpu.PrefetchScalarGridSpec(
        num_scalar_prefetch=1,
        grid=(NB,),
        in_specs=[
            pl.BlockSpec((B, D), lambda b, be_ref: (b, 0)),
            pl.BlockSpec((1, D, D), lambda b, be_ref: (be_ref[b], 0, 0)),
            pl.BlockSpec((1, 1, D), lambda b, be_ref: (be_ref[b], 0, 0)),
        ],
        out_specs=pl.BlockSpec((B, D), lambda b, be_ref: (b, 0)),
    )
    return pl.pallas_call(
        _expert_body,
        grid_spec=grid_spec,
        out_shape=jax.ShapeDtypeStruct((P, D), jnp.float32),
    )(blk_e, xs, We, be.reshape(E, 1, D))


_TPW = T // NW


def _gather2_body(ys_hbm, pos0_hbm, pos1_hbm, sel_hbm, i0, i1, r0, r1, s0, s1):
    wid = lax.axis_index("s") * NC + lax.axis_index("c")
    base = wid * _TPW
    pltpu.sync_copy(pos0_hbm.at[pl.ds(base, _TPW)], i0)
    pltpu.sync_copy(pos1_hbm.at[pl.ds(base, _TPW)], i1)
    c0 = pltpu.async_copy(ys_hbm.at[i0], r0, s0)
    c1 = pltpu.async_copy(ys_hbm.at[i1], r1, s1)
    c0.wait()
    c1.wait()
    pltpu.sync_copy(r0, sel_hbm.at[pl.ds(base, _TPW)])
    pltpu.sync_copy(r1, sel_hbm.at[pl.ds(T + base, _TPW)])


def _gather2(ys, pos0, pos1):
    return pl.kernel(
        _gather2_body,
        out_type=jax.ShapeDtypeStruct((2 * T, D), jnp.float32),
        mesh=plsc.VectorSubcoreMesh(**_SC_MESH),
        scratch_types=[
            pltpu.VMEM((_TPW,), jnp.int32),
            pltpu.VMEM((_TPW,), jnp.int32),
            pltpu.VMEM((_TPW, D), jnp.float32),
            pltpu.VMEM((_TPW, D), jnp.float32),
            pltpu.SemaphoreType.DMA,
            pltpu.SemaphoreType.DMA,
        ],
    )(ys, pos0, pos1)


_AR = 256


def _add_body(sel_ref, w_ref, out_ref):
    out_ref[...] = (sel_ref[0] * w_ref[0, 0, :][:, None]
                    + sel_ref[1] * w_ref[1, 0, :][:, None])


def _combine(ys, pos0, pos1, wgt):
    sel = _gather2(ys, pos0, pos1).reshape(2, T, D)
    return pl.pallas_call(
        _add_body,
        grid=(T // _AR,),
        in_specs=[
            pl.BlockSpec((2, _AR, D), lambda i: (0, i, 0)),
            pl.BlockSpec((2, 1, _AR), lambda i: (0, 0, i)),
        ],
        out_specs=pl.BlockSpec((_AR, D), lambda i: (i, 0)),
        out_shape=jax.ShapeDtypeStruct((T, D), jnp.float32),
    )(sel, wgt.reshape(2, 1, T))


def kernel(x, We, be, Wg, bg):
    pslot, wgt, blk_e, aux = _gating(x, Wg, bg)
    xs = _dispatch(x, pslot.reshape(T * K))
    ys = _expert_matmul(blk_e.reshape(NB), xs, We.astype(jnp.bfloat16), be)
    out = _combine(ys, pslot[0], pslot[1], wgt)
    return out, aux.reshape(())


# speedup vs baseline: 1.3922x; 1.3922x over previous
import functools

import jax
import jax.numpy as jnp
from jax import lax
from jax.experimental import pallas as pl
from jax.experimental.pallas import tpu as pltpu
from jax.experimental.pallas import tpu_sc as plsc

T, D, E, K = 2048, 768, 64, 2
B = 64
NB = T * K // B + E
P = NB * B

NC, NS = 2, 16
NW = NC * NS

_SC_MESH = dict(core_axis_name="c", subcore_axis_name="s",
                num_cores=NC, num_subcores=NS)


def _gating_body(x_ref, wg_ref, bg_ref, pslot_ref, w_ref, rix_ref, rexp_ref,
                 aux_ref):
    x = x_ref[...]
    logits = jnp.dot(x, wg_ref[...], preferred_element_type=jnp.float32)
    logits = logits + bg_ref[...]

    m0 = jnp.max(logits, axis=1, keepdims=True)
    ex = jnp.exp(logits - m0)
    gates = ex / jnp.sum(ex, axis=1, keepdims=True)
    imp = jnp.mean(gates, axis=0, keepdims=True)
    aux_ref[...] = jnp.sum((1.0 / E) * (jnp.log(1.0 / E) - jnp.log(imp)),
                           keepdims=True)

    lanes = lax.broadcasted_iota(jnp.int32, (T, E), 1)
    a0 = jnp.min(jnp.where(logits == m0, lanes, E), axis=1)
    masked = jnp.where(lanes == a0[:, None], -jnp.inf, logits)
    m1 = jnp.max(masked, axis=1, keepdims=True)
    a1 = jnp.min(jnp.where(masked == m1, lanes, E), axis=1)
    t = jnp.exp(m1 - m0)
    w0 = 1.0 / (1.0 + t)
    w_ref[0, :] = w0[:, 0]
    w_ref[1, :] = (t * w0)[:, 0]

    oh0 = (lanes == a0[:, None]).astype(jnp.float32)
    oh1 = (lanes == a1[:, None]).astype(jnp.float32)
    tri = (lax.broadcasted_iota(jnp.int32, (T, T), 0)
           > lax.broadcasted_iota(jnp.int32, (T, T), 1)).astype(jnp.float32)
    c0 = jnp.dot(tri, oh0, preferred_element_type=jnp.float32)
    c1 = jnp.dot(tri, oh1, preferred_element_type=jnp.float32)
    r0 = jnp.sum(c0 * oh0, axis=1)
    r1 = jnp.sum(c1 * oh1, axis=1)
    n0 = jnp.sum(oh0, axis=0, keepdims=True)
    n1 = jnp.sum(oh1, axis=0, keepdims=True)
    counts = (n0 + n1).astype(jnp.int32)
    pcounts = ((counts + (B - 1)) // B) * B
    etri = (lax.broadcasted_iota(jnp.int32, (E, E), 0)
            < lax.broadcasted_iota(jnp.int32, (E, E), 1)).astype(jnp.float32)
    poff = jnp.dot(pcounts.astype(jnp.float32), etri,
                   preferred_element_type=jnp.float32)
    poff_a0 = jnp.sum(poff * oh0, axis=1)
    poff_a1 = jnp.sum(poff * oh1, axis=1)
    n0_a1 = jnp.sum(n0 * oh1, axis=1)
    pslot_ref[0, :] = (poff_a0 + r0).astype(jnp.int32)
    pslot_ref[1, :] = (poff_a1 + n0_a1 + r1).astype(jnp.int32)

    bidx = (lax.broadcasted_iota(jnp.int32, (NB, E), 0) * B).astype(
        jnp.float32)
    blk = jnp.sum((poff <= bidx).astype(jnp.int32), axis=1) - 1
    ncross = jnp.sum(((poff > bidx - float(B)) & (poff <= bidx)).astype(
        jnp.float32), axis=1)
    chg = jnp.where((lax.broadcasted_iota(jnp.int32, (NB, 1), 0) > 0)[:, 0]
                    & (ncross >= 1.0), 1.0, 0.0).reshape(1, NB)
    btri = (lax.broadcasted_iota(jnp.int32, (NB, NB), 0)
            <= lax.broadcasted_iota(jnp.int32, (NB, NB), 1)).astype(
                jnp.float32)
    rix = jnp.dot(chg, btri, preferred_element_type=jnp.float32)
    rix_i = rix.astype(jnp.int32)
    rix_ref[0, :] = rix_i[0]
    run_oh = (rix_i[0][:, None]
              == lax.broadcasted_iota(jnp.int32, (NB, NB), 1))
    rexp_ref[0, :] = jnp.max(jnp.where(run_oh, blk[:, None], -1), axis=0)


def _gating(x, Wg, bg):
    return pl.pallas_call(
        _gating_body,
        out_shape=[
            jax.ShapeDtypeStruct((2, T), jnp.int32),
            jax.ShapeDtypeStruct((2, T), jnp.float32),
            jax.ShapeDtypeStruct((1, NB), jnp.int32),
            jax.ShapeDtypeStruct((1, NB), jnp.int32),
            jax.ShapeDtypeStruct((1, 1), jnp.float32),
        ],
    )(x, Wg, bg.reshape(1, E))


_APW = T * K // NW


def _dispatch_body(x_hbm, pslot_hbm, xs_hbm, idx_v, rows_v, sem):
    wid = lax.axis_index("s") * NC + lax.axis_index("c")
    base = wid * _APW
    tok0 = pl.multiple_of(base & (T - 1), 8)
    pltpu.sync_copy(pslot_hbm.at[pl.ds(base, _APW)], idx_v.at[0])
    pltpu.sync_copy(x_hbm.at[pl.ds(tok0, _APW)], rows_v)
    pltpu.async_copy(rows_v, xs_hbm.at[idx_v.at[0]], sem).wait()


def _dispatch(x, pslot_flat):
    return pl.kernel(
        _dispatch_body,
        out_type=jax.ShapeDtypeStruct((P, D), jnp.float32),
        mesh=plsc.VectorSubcoreMesh(**_SC_MESH),
        scratch_types=[
            pltpu.VMEM((1, _APW), jnp.int32),
            pltpu.VMEM((_APW, D), jnp.float32),
            pltpu.SemaphoreType.DMA,
        ],
    )(x, pslot_flat)


_NWB = 3


def _expert_body(rix_ref, rexp_ref, xs_ref, we_hbm, be_ref, ys_ref,
                 we_buf, sems):
    b = pl.program_id(0)
    r = rix_ref[b]
    rmax = rix_ref[NB - 1]
    first = jnp.logical_or(b == 0, r != rix_ref[jnp.maximum(b - 1, 0)])

    def _issue(rr):
        pltpu.make_async_copy(we_hbm.at[rexp_ref[rr]],
                              we_buf.at[rr % _NWB],
                              sems.at[rr % _NWB]).start()

    @pl.when(b == 0)
    def _():
        _issue(0)

    @pl.when(jnp.logical_and(b == 0, rmax >= 1))
    def _():
        _issue(1)

    @pl.when(jnp.logical_and(b == 0, rmax >= 2))
    def _():
        _issue(2)

    @pl.when(jnp.logical_and(first,
                             jnp.logical_and(b > 0, r + 2 <= rmax)))
    def _():
        _issue(r + 2)

    @pl.when(first)
    def _():
        pltpu.make_async_copy(we_hbm.at[rexp_ref[r]],
                              we_buf.at[r % _NWB],
                              sems.at[r % _NWB]).wait()

    acc = jnp.dot(xs_ref[...], we_buf[r % _NWB],
                  preferred_element_type=jnp.float32)
    ys_ref[...] = acc + be_ref[0]


def _expert_matmul(rix, rexp, xs, We, be):
    grid_spec = pltpu.PrefetchScalarGridSpec(
        num_scalar_prefetch=2,
        grid=(NB,),
        in_specs=[
            pl.BlockSpec((B, D), lambda b, rix_r, rexp_r: (b, 0)),
            pl.BlockSpec(memory_space=pl.ANY),
            pl.BlockSpec((1, 1, D),
                         lambda b, rix_r, rexp_r: (rexp_r[rix_r[b]], 0, 0)),
        ],
        out_specs=pl.BlockSpec((B, D), lambda b, rix_r, rexp_r: (b, 0)),
        scratch_shapes=[
            pltpu.VMEM((_NWB, D, D), jnp.float32),
            pltpu.SemaphoreType.DMA((_NWB,)),
        ],
    )
    return pl.pallas_call(
        _expert_body,
        grid_spec=grid_spec,
        out_shape=jax.ShapeDtypeStruct((P, D), jnp.float32),
    )(rix, rexp, xs, We, be.reshape(E, 1, D))


_TPW = T // NW


def _gather2_body(ys_hbm, pos0_hbm, pos1_hbm, sel_hbm, i0, i1, r0, r1, s0, s1):
    wid = lax.axis_index("s") * NC + lax.axis_index("c")
    base = wid * _TPW
    pltpu.sync_copy(pos0_hbm.at[pl.ds(base, _TPW)], i0)
    pltpu.sync_copy(pos1_hbm.at[pl.ds(base, _TPW)], i1)
    c0 = pltpu.async_copy(ys_hbm.at[i0], r0, s0)
    c1 = pltpu.async_copy(ys_hbm.at[i1], r1, s1)
    c0.wait()
    c1.wait()
    pltpu.sync_copy(r0, sel_hbm.at[pl.ds(base, _TPW)])
    pltpu.sync_copy(r1, sel_hbm.at[pl.ds(T + base, _TPW)])


def _gather2(ys, pos0, pos1):
    return pl.kernel(
        _gather2_body,
        out_type=jax.ShapeDtypeStruct((2 * T, D), jnp.float32),
        mesh=plsc.VectorSubcoreMesh(**_SC_MESH),
        scratch_types=[
            pltpu.VMEM((_TPW,), jnp.int32),
            pltpu.VMEM((_TPW,), jnp.int32),
            pltpu.VMEM((_TPW, D), jnp.float32),
            pltpu.VMEM((_TPW, D), jnp.float32),
            pltpu.SemaphoreType.DMA,
            pltpu.SemaphoreType.DMA,
        ],
    )(ys, pos0, pos1)


_AR = 256


def _add_body(sel_ref, w_ref, out_ref):
    out_ref[...] = (sel_ref[0] * w_ref[0, 0, :][:, None]
                    + sel_ref[1] * w_ref[1, 0, :][:, None])


def _combine(ys, pos0, pos1, wgt):
    sel = _gather2(ys, pos0, pos1).reshape(2, T, D)
    return pl.pallas_call(
        _add_body,
        grid=(T // _AR,),
        in_specs=[
            pl.BlockSpec((2, _AR, D), lambda i: (0, i, 0)),
            pl.BlockSpec((2, 1, _AR), lambda i: (0, 0, i)),
        ],
        out_specs=pl.BlockSpec((_AR, D), lambda i: (i, 0)),
        out_shape=jax.ShapeDtypeStruct((T, D), jnp.float32),
    )(sel, wgt.reshape(2, 1, T))


def kernel(x, We, be, Wg, bg):
    pslot, wgt, rix, rexp, aux = _gating(x, Wg, bg)
    xs = _dispatch(x, pslot.reshape(T * K))
    ys = _expert_matmul(rix.reshape(NB), rexp.reshape(NB), xs, We, be)
    out = _combine(ys, pslot[0], pslot[1], wgt)
    return out, aux.reshape(())


# speedup vs baseline: 1.3924x; 1.0002x over previous
import functools

import jax
import jax.numpy as jnp
from jax import lax
from jax.experimental import pallas as pl
from jax.experimental.pallas import tpu as pltpu
from jax.experimental.pallas import tpu_sc as plsc

T, D, E, K = 2048, 768, 64, 2
B = 64
NB = T * K // B + E
P = NB * B

NC, NS = 2, 16
NW = NC * NS

_SC_MESH = dict(core_axis_name="c", subcore_axis_name="s",
                num_cores=NC, num_subcores=NS)


def _gating_body(x_ref, wg_ref, bg_ref, pslot_ref, w_ref, rix_ref, rexp_ref,
                 aux_ref):
    x = x_ref[...]
    logits = jnp.dot(x, wg_ref[...], preferred_element_type=jnp.float32)
    logits = logits + bg_ref[...]

    m0 = jnp.max(logits, axis=1, keepdims=True)
    ex = jnp.exp(logits - m0)
    gates = ex / jnp.sum(ex, axis=1, keepdims=True)
    imp = jnp.mean(gates, axis=0, keepdims=True)
    aux_ref[...] = jnp.sum((1.0 / E) * (jnp.log(1.0 / E) - jnp.log(imp)),
                           keepdims=True)

    lanes = lax.broadcasted_iota(jnp.int32, (T, E), 1)
    a0 = jnp.min(jnp.where(logits == m0, lanes, E), axis=1)
    masked = jnp.where(lanes == a0[:, None], -jnp.inf, logits)
    m1 = jnp.max(masked, axis=1, keepdims=True)
    a1 = jnp.min(jnp.where(masked == m1, lanes, E), axis=1)
    t = jnp.exp(m1 - m0)
    w0 = 1.0 / (1.0 + t)
    w_ref[0, :] = w0[:, 0]
    w_ref[1, :] = (t * w0)[:, 0]

    oh0 = (lanes == a0[:, None]).astype(jnp.float32)
    oh1 = (lanes == a1[:, None]).astype(jnp.float32)
    tri = (lax.broadcasted_iota(jnp.int32, (T, T), 0)
           > lax.broadcasted_iota(jnp.int32, (T, T), 1)).astype(jnp.float32)
    c0 = jnp.dot(tri, oh0, preferred_element_type=jnp.float32)
    c1 = jnp.dot(tri, oh1, preferred_element_type=jnp.float32)
    r0 = jnp.sum(c0 * oh0, axis=1)
    r1 = jnp.sum(c1 * oh1, axis=1)
    n0 = jnp.sum(oh0, axis=0, keepdims=True)
    n1 = jnp.sum(oh1, axis=0, keepdims=True)
    counts = (n0 + n1).astype(jnp.int32)
    pcounts = ((counts + (B - 1)) // B) * B
    etri = (lax.broadcasted_iota(jnp.int32, (E, E), 0)
            < lax.broadcasted_iota(jnp.int32, (E, E), 1)).astype(jnp.float32)
    poff = jnp.dot(pcounts.astype(jnp.float32), etri,
                   preferred_element_type=jnp.float32)
    poff_a0 = jnp.sum(poff * oh0, axis=1)
    poff_a1 = jnp.sum(poff * oh1, axis=1)
    n0_a1 = jnp.sum(n0 * oh1, axis=1)
    pslot_ref[0, :] = (poff_a0 + r0).astype(jnp.int32)
    pslot_ref[1, :] = (poff_a1 + n0_a1 + r1).astype(jnp.int32)

    bidx = (lax.broadcasted_iota(jnp.int32, (NB, E), 0) * B).astype(
        jnp.float32)
    blk = jnp.sum((poff <= bidx).astype(jnp.int32), axis=1) - 1
    ncross = jnp.sum(((poff > bidx - float(B)) & (poff <= bidx)).astype(
        jnp.float32), axis=1)
    chg = jnp.where((lax.broadcasted_iota(jnp.int32, (NB, 1), 0) > 0)[:, 0]
                    & (ncross >= 1.0), 1.0, 0.0).reshape(1, NB)
    btri = (lax.broadcasted_iota(jnp.int32, (NB, NB), 0)
            <= lax.broadcasted_iota(jnp.int32, (NB, NB), 1)).astype(
                jnp.float32)
    rix = jnp.dot(chg, btri, preferred_element_type=jnp.float32)
    rix_i = rix.astype(jnp.int32)
    rix_ref[0, :] = rix_i[0]
    run_oh = (rix_i[0][:, None]
              == lax.broadcasted_iota(jnp.int32, (NB, NB), 1))
    rexp_ref[0, :] = jnp.max(jnp.where(run_oh, blk[:, None], -1), axis=0)


def _gating(x, Wg, bg):
    return pl.pallas_call(
        _gating_body,
        out_shape=[
            jax.ShapeDtypeStruct((2, T), jnp.int32),
            jax.ShapeDtypeStruct((2, T), jnp.float32),
            jax.ShapeDtypeStruct((1, NB), jnp.int32),
            jax.ShapeDtypeStruct((1, NB), jnp.int32),
            jax.ShapeDtypeStruct((1, 1), jnp.float32),
        ],
    )(x, Wg, bg.reshape(1, E))


_APW = T * K // NW


def _dispatch_body(x_hbm, pslot_hbm, xs_hbm, idx_v, rows_v, sem):
    wid = lax.axis_index("s") * NC + lax.axis_index("c")
    base = wid * _APW
    tok0 = pl.multiple_of(base & (T - 1), 8)
    pltpu.sync_copy(pslot_hbm.at[pl.ds(base, _APW)], idx_v.at[0])
    pltpu.sync_copy(x_hbm.at[pl.ds(tok0, _APW)], rows_v)
    pltpu.async_copy(rows_v, xs_hbm.at[idx_v.at[0]], sem).wait()


def _dispatch(x, pslot_flat):
    return pl.kernel(
        _dispatch_body,
        out_type=jax.ShapeDtypeStruct((P, D), jnp.float32),
        mesh=plsc.VectorSubcoreMesh(**_SC_MESH),
        scratch_types=[
            pltpu.VMEM((1, _APW), jnp.int32),
            pltpu.VMEM((_APW, D), jnp.float32),
            pltpu.SemaphoreType.DMA,
        ],
    )(x, pslot_flat)


_NWB = 4


def _expert_body(rix_ref, rexp_ref, xs_ref, we_hbm, be_ref, ys_ref,
                 we_buf, sems):
    b = pl.program_id(0)
    r = rix_ref[b]
    rmax = rix_ref[NB - 1]
    first = jnp.logical_or(b == 0, r != rix_ref[jnp.maximum(b - 1, 0)])

    def _issue(rr):
        pltpu.make_async_copy(we_hbm.at[rexp_ref[rr]],
                              we_buf.at[rr % _NWB],
                              sems.at[rr % _NWB]).start()

    @pl.when(b == 0)
    def _():
        _issue(0)

    @pl.when(jnp.logical_and(b == 0, rmax >= 1))
    def _():
        _issue(1)

    @pl.when(jnp.logical_and(b == 0, rmax >= 2))
    def _():
        _issue(2)

    @pl.when(jnp.logical_and(b == 0, rmax >= 3))
    def _():
        _issue(3)

    @pl.when(jnp.logical_and(first,
                             jnp.logical_and(b > 0, r + 3 <= rmax)))
    def _():
        _issue(r + 3)

    @pl.when(first)
    def _():
        pltpu.make_async_copy(we_hbm.at[rexp_ref[r]],
                              we_buf.at[r % _NWB],
                              sems.at[r % _NWB]).wait()

    acc = jnp.dot(xs_ref[...], we_buf[r % _NWB],
                  preferred_element_type=jnp.float32)
    ys_ref[...] = acc + be_ref[0]


def _expert_matmul(rix, rexp, xs, We, be):
    grid_spec = pltpu.PrefetchScalarGridSpec(
        num_scalar_prefetch=2,
        grid=(NB,),
        in_specs=[
            pl.BlockSpec((B, D), lambda b, rix_r, rexp_r: (b, 0)),
            pl.BlockSpec(memory_space=pl.ANY),
            pl.BlockSpec((1, 1, D),
                         lambda b, rix_r, rexp_r: (rexp_r[rix_r[b]], 0, 0)),
        ],
        out_specs=pl.BlockSpec((B, D), lambda b, rix_r, rexp_r: (b, 0)),
        scratch_shapes=[
            pltpu.VMEM((_NWB, D, D), jnp.float32),
            pltpu.SemaphoreType.DMA((_NWB,)),
        ],
    )
    return pl.pallas_call(
        _expert_body,
        grid_spec=grid_spec,
        out_shape=jax.ShapeDtypeStruct((P, D), jnp.float32),
    )(rix, rexp, xs, We, be.reshape(E, 1, D))


_TPW = T // NW


def _gather2_body(ys_hbm, pos0_hbm, pos1_hbm, sel_hbm, i0, i1, r0, r1, s0, s1):
    wid = lax.axis_index("s") * NC + lax.axis_index("c")
    base = wid * _TPW
    pltpu.sync_copy(pos0_hbm.at[pl.ds(base, _TPW)], i0)
    pltpu.sync_copy(pos1_hbm.at[pl.ds(base, _TPW)], i1)
    c0 = pltpu.async_copy(ys_hbm.at[i0], r0, s0)
    c1 = pltpu.async_copy(ys_hbm.at[i1], r1, s1)
    c0.wait()
    c1.wait()
    pltpu.sync_copy(r0, sel_hbm.at[pl.ds(base, _TPW)])
    pltpu.sync_copy(r1, sel_hbm.at[pl.ds(T + base, _TPW)])


def _gather2(ys, pos0, pos1):
    return pl.kernel(
        _gather2_body,
        out_type=jax.ShapeDtypeStruct((2 * T, D), jnp.float32),
        mesh=plsc.VectorSubcoreMesh(**_SC_MESH),
        scratch_types=[
            pltpu.VMEM((_TPW,), jnp.int32),
            pltpu.VMEM((_TPW,), jnp.int32),
            pltpu.VMEM((_TPW, D), jnp.float32),
            pltpu.VMEM((_TPW, D), jnp.float32),
            pltpu.SemaphoreType.DMA,
            pltpu.SemaphoreType.DMA,
        ],
    )(ys, pos0, pos1)


_AR = 256


def _add_body(sel_ref, w_ref, out_ref):
    out_ref[...] = (sel_ref[0] * w_ref[0, 0, :][:, None]
                    + sel_ref[1] * w_ref[1, 0, :][:, None])


def _combine(ys, pos0, pos1, wgt):
    sel = _gather2(ys, pos0, pos1).reshape(2, T, D)
    return pl.pallas_call(
        _add_body,
        grid=(T // _AR,),
        in_specs=[
            pl.BlockSpec((2, _AR, D), lambda i: (0, i, 0)),
            pl.BlockSpec((2, 1, _AR), lambda i: (0, 0, i)),
        ],
        out_specs=pl.BlockSpec((_AR, D), lambda i: (i, 0)),
        out_shape=jax.ShapeDtypeStruct((T, D), jnp.float32),
    )(sel, wgt.reshape(2, 1, T))


def kernel(x, We, be, Wg, bg):
    pslot, wgt, rix, rexp, aux = _gating(x, Wg, bg)
    xs = _dispatch(x, pslot.reshape(T * K))
    ys = _expert_matmul(rix.reshape(NB), rexp.reshape(NB), xs, We, be)
    out = _combine(ys, pslot[0], pslot[1], wgt)
    return out, aux.reshape(())


# speedup vs baseline: 1.3928x; 1.0002x over previous
import functools

import jax
import jax.numpy as jnp
from jax import lax
from jax.experimental import pallas as pl
from jax.experimental.pallas import tpu as pltpu
from jax.experimental.pallas import tpu_sc as plsc

T, D, E, K = 2048, 768, 64, 2
B = 64
NB = T * K // B + E
P = NB * B

NC, NS = 2, 16
NW = NC * NS

_SC_MESH = dict(core_axis_name="c", subcore_axis_name="s",
                num_cores=NC, num_subcores=NS)


def _gating_body(x_ref, wg_ref, bg_ref, pslot_ref, w_ref, rix_ref, rexp_ref,
                 aux_ref):
    x = x_ref[...]
    logits = jnp.dot(x, wg_ref[...], preferred_element_type=jnp.float32)
    logits = logits + bg_ref[...]

    m0 = jnp.max(logits, axis=1, keepdims=True)
    ex = jnp.exp(logits - m0)
    gates = ex / jnp.sum(ex, axis=1, keepdims=True)
    imp = jnp.mean(gates, axis=0, keepdims=True)
    aux_ref[...] = jnp.sum((1.0 / E) * (jnp.log(1.0 / E) - jnp.log(imp)),
                           keepdims=True)

    lanes = lax.broadcasted_iota(jnp.int32, (T, E), 1)
    a0 = jnp.min(jnp.where(logits == m0, lanes, E), axis=1)
    masked = jnp.where(lanes == a0[:, None], -jnp.inf, logits)
    m1 = jnp.max(masked, axis=1, keepdims=True)
    a1 = jnp.min(jnp.where(masked == m1, lanes, E), axis=1)
    t = jnp.exp(m1 - m0)
    w0 = 1.0 / (1.0 + t)
    w_ref[0, :] = w0[:, 0]
    w_ref[1, :] = (t * w0)[:, 0]

    oh0 = (lanes == a0[:, None]).astype(jnp.float32)
    oh1 = (lanes == a1[:, None]).astype(jnp.float32)
    tri = (lax.broadcasted_iota(jnp.int32, (T, T), 0)
           > lax.broadcasted_iota(jnp.int32, (T, T), 1)).astype(jnp.float32)
    c0 = jnp.dot(tri, oh0, preferred_element_type=jnp.float32)
    c1 = jnp.dot(tri, oh1, preferred_element_type=jnp.float32)
    r0 = jnp.sum(c0 * oh0, axis=1)
    r1 = jnp.sum(c1 * oh1, axis=1)
    n0 = jnp.sum(oh0, axis=0, keepdims=True)
    n1 = jnp.sum(oh1, axis=0, keepdims=True)
    counts = (n0 + n1).astype(jnp.int32)
    pcounts = ((counts + (B - 1)) // B) * B
    etri = (lax.broadcasted_iota(jnp.int32, (E, E), 0)
            < lax.broadcasted_iota(jnp.int32, (E, E), 1)).astype(jnp.float32)
    poff = jnp.dot(pcounts.astype(jnp.float32), etri,
                   preferred_element_type=jnp.float32)
    poff_a0 = jnp.sum(poff * oh0, axis=1)
    poff_a1 = jnp.sum(poff * oh1, axis=1)
    n0_a1 = jnp.sum(n0 * oh1, axis=1)
    pslot_ref[0, :] = (poff_a0 + r0).astype(jnp.int32)
    pslot_ref[1, :] = (poff_a1 + n0_a1 + r1).astype(jnp.int32)

    bidx = (lax.broadcasted_iota(jnp.int32, (NB, E), 0) * B).astype(
        jnp.float32)
    blk = jnp.sum((poff <= bidx).astype(jnp.int32), axis=1) - 1
    ncross = jnp.sum(((poff > bidx - float(B)) & (poff <= bidx)).astype(
        jnp.float32), axis=1)
    chg = jnp.where((lax.broadcasted_iota(jnp.int32, (NB, 1), 0) > 0)[:, 0]
                    & (ncross >= 1.0), 1.0, 0.0).reshape(1, NB)
    btri = (lax.broadcasted_iota(jnp.int32, (NB, NB), 0)
            <= lax.broadcasted_iota(jnp.int32, (NB, NB), 1)).astype(
                jnp.float32)
    rix = jnp.dot(chg, btri, preferred_element_type=jnp.float32)
    rix_i = rix.astype(jnp.int32)
    rix_ref[0, :] = rix_i[0]
    run_oh = (rix_i[0][:, None]
              == lax.broadcasted_iota(jnp.int32, (NB, NB), 1))
    rexp_ref[0, :] = jnp.max(jnp.where(run_oh, blk[:, None], -1), axis=0)


def _gating(x, Wg, bg):
    return pl.pallas_call(
        _gating_body,
        out_shape=[
            jax.ShapeDtypeStruct((2, T), jnp.int32),
            jax.ShapeDtypeStruct((2, T), jnp.float32),
            jax.ShapeDtypeStruct((1, NB), jnp.int32),
            jax.ShapeDtypeStruct((1, NB), jnp.int32),
            jax.ShapeDtypeStruct((1, 1), jnp.float32),
        ],
    )(x, Wg, bg.reshape(1, E))


_APW = T * K // NW


def _dispatch_body(x_hbm, pslot_hbm, xs_hbm, idx_v, rows_v, sem):
    wid = lax.axis_index("s") * NC + lax.axis_index("c")
    base = wid * _APW
    tok0 = pl.multiple_of(base & (T - 1), 8)
    pltpu.sync_copy(pslot_hbm.at[pl.ds(base, _APW)], idx_v.at[0])
    pltpu.sync_copy(x_hbm.at[pl.ds(tok0, _APW)], rows_v)
    pltpu.async_copy(rows_v, xs_hbm.at[idx_v.at[0]], sem).wait()


def _dispatch(x, pslot_flat):
    return pl.kernel(
        _dispatch_body,
        out_type=jax.ShapeDtypeStruct((P, D), jnp.float32),
        mesh=plsc.VectorSubcoreMesh(**_SC_MESH),
        scratch_types=[
            pltpu.VMEM((1, _APW), jnp.int32),
            pltpu.VMEM((_APW, D), jnp.float32),
            pltpu.SemaphoreType.DMA,
        ],
    )(x, pslot_flat)


_NWB = 3


def _expert_body(rix_ref, rexp_ref, xs_ref, we_hbm, be_ref, ys_ref,
                 we_buf, sems):
    b = pl.program_id(0)
    r = rix_ref[b]
    rmax = rix_ref[NB - 1]
    first = jnp.logical_or(b == 0, r != rix_ref[jnp.maximum(b - 1, 0)])

    def _issue(rr):
        pltpu.make_async_copy(we_hbm.at[rexp_ref[rr]],
                              we_buf.at[rr % _NWB],
                              sems.at[rr % _NWB]).start()

    @pl.when(b == 0)
    def _():
        _issue(0)

    @pl.when(jnp.logical_and(b == 0, rmax >= 1))
    def _():
        _issue(1)

    @pl.when(jnp.logical_and(b == 0, rmax >= 2))
    def _():
        _issue(2)

    @pl.when(jnp.logical_and(first,
                             jnp.logical_and(b > 0, r + 2 <= rmax)))
    def _():
        _issue(r + 2)

    @pl.when(first)
    def _():
        pltpu.make_async_copy(we_hbm.at[rexp_ref[r]],
                              we_buf.at[r % _NWB],
                              sems.at[r % _NWB]).wait()

    acc = jnp.dot(xs_ref[...], we_buf[r % _NWB],
                  preferred_element_type=jnp.float32)
    ys_ref[...] = acc + be_ref[0]


def _expert_matmul(rix, rexp, xs, We, be):
    grid_spec = pltpu.PrefetchScalarGridSpec(
        num_scalar_prefetch=2,
        grid=(NB,),
        in_specs=[
            pl.BlockSpec((B, D), lambda b, rix_r, rexp_r: (b, 0)),
            pl.BlockSpec(memory_space=pl.ANY),
            pl.BlockSpec((1, 1, D),
                         lambda b, rix_r, rexp_r: (rexp_r[rix_r[b]], 0, 0)),
        ],
        out_specs=pl.BlockSpec((B, D), lambda b, rix_r, rexp_r: (b, 0)),
        scratch_shapes=[
            pltpu.VMEM((_NWB, D, D), jnp.float32),
            pltpu.SemaphoreType.DMA((_NWB,)),
        ],
    )
    return pl.pallas_call(
        _expert_body,
        grid_spec=grid_spec,
        out_shape=jax.ShapeDtypeStruct((P, D), jnp.float32),
    )(rix, rexp, xs, We, be.reshape(E, 1, D))


_TPW = T // NW


def _gather2_body(ys_hbm, pos0_hbm, pos1_hbm, sel_hbm, i0, i1, r0, r1, s0, s1):
    wid = lax.axis_index("s") * NC + lax.axis_index("c")
    base = wid * _TPW
    pltpu.sync_copy(pos0_hbm.at[pl.ds(base, _TPW)], i0)
    pltpu.sync_copy(pos1_hbm.at[pl.ds(base, _TPW)], i1)
    c0 = pltpu.async_copy(ys_hbm.at[i0], r0, s0)
    c1 = pltpu.async_copy(ys_hbm.at[i1], r1, s1)
    c0.wait()
    c1.wait()
    pltpu.sync_copy(r0, sel_hbm.at[pl.ds(base, _TPW)])
    pltpu.sync_copy(r1, sel_hbm.at[pl.ds(T + base, _TPW)])


def _gather2(ys, pos0, pos1):
    return pl.kernel(
        _gather2_body,
        out_type=jax.ShapeDtypeStruct((2 * T, D), jnp.float32),
        mesh=plsc.VectorSubcoreMesh(**_SC_MESH),
        scratch_types=[
            pltpu.VMEM((_TPW,), jnp.int32),
            pltpu.VMEM((_TPW,), jnp.int32),
            pltpu.VMEM((_TPW, D), jnp.float32),
            pltpu.VMEM((_TPW, D), jnp.float32),
            pltpu.SemaphoreType.DMA,
            pltpu.SemaphoreType.DMA,
        ],
    )(ys, pos0, pos1)


_AR = 256


def _add_body(sel_ref, w_ref, out_ref):
    out_ref[...] = (sel_ref[0] * w_ref[0, 0, :][:, None]
                    + sel_ref[1] * w_ref[1, 0, :][:, None])


def _combine(ys, pos0, pos1, wgt):
    sel = _gather2(ys, pos0, pos1).reshape(2, T, D)
    return pl.pallas_call(
        _add_body,
        grid=(T // _AR,),
        in_specs=[
            pl.BlockSpec((2, _AR, D), lambda i: (0, i, 0)),
            pl.BlockSpec((2, 1, _AR), lambda i: (0, 0, i)),
        ],
        out_specs=pl.BlockSpec((_AR, D), lambda i: (i, 0)),
        out_shape=jax.ShapeDtypeStruct((T, D), jnp.float32),
    )(sel, wgt.reshape(2, 1, T))


def kernel(x, We, be, Wg, bg):
    pslot, wgt, rix, rexp, aux = _gating(x, Wg, bg)
    xs = _dispatch(x, pslot.reshape(T * K))
    ys = _expert_matmul(rix.reshape(NB), rexp.reshape(NB), xs, We, be)
    out = _combine(ys, pslot[0], pslot[1], wgt)
    return out, aux.reshape(())
